# Initial kernel scaffold; baseline (speedup 1.0000x reference)
import jax
import jax.numpy as jnp
from jax.experimental import pallas as pl


def kernel(mem, items_ids, items_memory):
    raise NotImplementedError("write your pallas kernel here")


# speedup vs baseline: 1.1214x; 1.1214x over previous
import jax
import jax.numpy as jnp
from jax import lax
from jax.experimental import pallas as pl
from jax.experimental.pallas import tpu as pltpu
from jax.experimental.pallas import tpu_sc as plsc

NUM_ITEMS = 100000
MEM_DIM = 64
BATCH = 16384
ALPHA = 0.5

NC = 2
NS = 16
NW = NC * NS
ITEMS_PER_TILE = BATCH // NW
IDS_ROWS_PER_TILE = ITEMS_PER_TILE // 128
HALF_IDS = NUM_ITEMS // 2
P_BITS = 14
P_MASK = (1 << P_BITS) - 1

_MESH = plsc.VectorSubcoreMesh(
    core_axis_name="c", subcore_axis_name="s", num_cores=NC, num_subcores=NS
)
_PARAMS = pltpu.CompilerParams(
    needs_layout_passes=False, use_tc_tiling_on_sc=False
)


def _blend_phase(out_hbm, ids2d, items_memory):
  @pl.kernel(
      out_type=jax.ShapeDtypeStruct((BATCH, MEM_DIM), jnp.float32),
      mesh=_MESH,
      compiler_params=_PARAMS,
      scratch_types=[
          pltpu.VMEM((HALF_IDS,), jnp.int32),
          pltpu.VMEM((32, 128), jnp.int32),
          pltpu.VMEM((IDS_ROWS_PER_TILE, 128), jnp.int32),
          pltpu.VMEM((IDS_ROWS_PER_TILE, 128), jnp.int32),
          pltpu.VMEM((IDS_ROWS_PER_TILE, 128), jnp.int32),
          pltpu.VMEM((ITEMS_PER_TILE, MEM_DIM), jnp.float32),
          pltpu.VMEM((ITEMS_PER_TILE, MEM_DIM), jnp.float32),
          pltpu.VMEM_SHARED((NUM_ITEMS,), jnp.int32),
      ],
  )
  def k1(out_hbm, ids_hbm, upd_hbm, blended_hbm,
         win_v, idsb_v, idx_v, kwin_v, q_v, rows_v, upd_v, win_sp):
    c = lax.axis_index("c")
    s = lax.axis_index("s")
    wid = c * NS + s
    iota16 = lax.iota(jnp.int32, 16)

    @pl.when(s < 2)
    def _build():
      lo = s * HALF_IDS
      for r in range(4):
        pltpu.sync_copy(ids_hbm.at[pl.ds(r * 32, 32)], idsb_v)

        def row_body(row, _):
          for kblk in range(8):
            idv = idsb_v[row, pl.ds(kblk * 16, 16)]
            p0 = (r * 32 + row) * 128 + kblk * 16
            kkey = (idv << P_BITS) | (iota16 + p0)
            idl = jnp.clip(idv - lo, 0, HALF_IDS - 1)
            m_in = (idv >= lo) & (idv < lo + HALF_IDS)
            plsc.store_scatter(win_v, [idl], kkey, mask=m_in)
            r0 = plsc.load_gather(win_v, [idl], mask=m_in)
            m = m_in & (kkey > r0)

            def cond(carry):
              return jnp.any(carry[0])

            def body(carry):
              (mm,) = carry
              plsc.store_scatter(win_v, [idl], kkey, mask=mm)
              r1 = plsc.load_gather(win_v, [idl], mask=m_in)
              return (m_in & (kkey > r1),)

            lax.while_loop(cond, body, (m,))
          return 0

        lax.fori_loop(0, 32, row_body, 0)
      pltpu.sync_copy(win_v, win_sp.at[pl.ds(lo, HALF_IDS)])

    pltpu.sync_copy(
        ids_hbm.at[pl.ds(wid * IDS_ROWS_PER_TILE, IDS_ROWS_PER_TILE)], idx_v
    )
    for j in range(IDS_ROWS_PER_TILE):
      pltpu.sync_copy(out_hbm.at[idx_v.at[j]], rows_v.at[pl.ds(j * 128, 128)])

    plsc.subcore_barrier()

    for j in range(IDS_ROWS_PER_TILE):
      pltpu.sync_copy(win_sp.at[idx_v.at[j]], kwin_v.at[j])
    for j in range(IDS_ROWS_PER_TILE):
      for kblk in range(8):
        q_v[j, pl.ds(kblk * 16, 16)] = kwin_v[j, pl.ds(kblk * 16, 16)] & P_MASK
    for j in range(IDS_ROWS_PER_TILE):
      pltpu.sync_copy(upd_hbm.at[q_v.at[j]], upd_v.at[pl.ds(j * 128, 128)])

    def blend_body(i, _):
      for kblk in range(MEM_DIM // 16):
        sl = pl.ds(kblk * 16, 16)
        rows_v[i, sl] = (rows_v[i, sl] + upd_v[i, sl]) * ALPHA
      return 0

    lax.fori_loop(0, ITEMS_PER_TILE, blend_body, 0)

    pltpu.sync_copy(
        rows_v, blended_hbm.at[pl.ds(wid * ITEMS_PER_TILE, ITEMS_PER_TILE)]
    )

  return k1(out_hbm, ids2d, items_memory)


def _scatter_phase(out_ref, ids2d, blended):
  @pl.kernel(
      out_type=(),
      mesh=_MESH,
      compiler_params=_PARAMS,
      scratch_types=[
          pltpu.VMEM((IDS_ROWS_PER_TILE, 128), jnp.int32),
          pltpu.VMEM((ITEMS_PER_TILE, MEM_DIM), jnp.float32),
      ],
  )
  def k2(out_hbm, ids_hbm, blended_hbm, idx_v, rows_v):
    c = lax.axis_index("c")
    s = lax.axis_index("s")
    wid = c * NS + s
    pltpu.sync_copy(
        ids_hbm.at[pl.ds(wid * IDS_ROWS_PER_TILE, IDS_ROWS_PER_TILE)], idx_v
    )
    pltpu.sync_copy(
        blended_hbm.at[pl.ds(wid * ITEMS_PER_TILE, ITEMS_PER_TILE)], rows_v
    )
    for j in range(IDS_ROWS_PER_TILE):
      pltpu.sync_copy(rows_v.at[pl.ds(j * 128, 128)], out_hbm.at[idx_v.at[j]])

  k2(out_ref, ids2d, blended)


def kernel(mem, items_ids, items_memory):
  ids2d = items_ids.astype(jnp.int32).reshape(128, 128)
  out_ref = jax.new_ref(mem)
  blended = _blend_phase(out_ref, ids2d, items_memory)
  _scatter_phase(out_ref, ids2d, blended)
  return jax.freeze(out_ref)


# speedup vs baseline: 1.1469x; 1.0227x over previous
import jax
import jax.numpy as jnp
from jax import lax
from jax.experimental import pallas as pl
from jax.experimental.pallas import tpu as pltpu
from jax.experimental.pallas import tpu_sc as plsc

NUM_ITEMS = 100000
MEM_DIM = 64
BATCH = 16384
ALPHA = 0.5

NC = 2
NS = 16
NW = NC * NS
P_BITS = 14
P_MASK = (1 << P_BITS) - 1
IDS_PER_TILE = NUM_ITEMS // NS
WIN_PAD = 6256
LIST_STRIDE = 7168
LISTS_LEN = NS * LIST_STRIDE
CHUNK = 4 * IDS_PER_TILE
NCHUNK = NUM_ITEMS // CHUNK
ROWS_PER_W = MEM_DIM // NW
PIECE = 1024

_MESH = plsc.VectorSubcoreMesh(
    core_axis_name="c", subcore_axis_name="s", num_cores=NC, num_subcores=NS
)
_PARAMS = pltpu.CompilerParams(
    needs_layout_passes=False, use_tc_tiling_on_sc=False
)


@pl.kernel(
    out_type=jax.ShapeDtypeStruct((MEM_DIM, NUM_ITEMS), jnp.float32),
    mesh=_MESH,
    compiler_params=_PARAMS,
    scratch_types=[
        pltpu.VMEM((WIN_PAD,), jnp.int32),
        pltpu.VMEM((4096,), jnp.int32),
        pltpu.VMEM((LIST_STRIDE,), jnp.int32),
        pltpu.VMEM((LIST_STRIDE,), jnp.int32),
        pltpu.VMEM((16,), jnp.int32),
        pltpu.VMEM((144,), jnp.int32),
        pltpu.VMEM((CHUNK,), jnp.float32),
        pltpu.VMEM((BATCH,), jnp.float32),
        pltpu.VMEM((BATCH,), jnp.float32),
        pltpu.VMEM((PIECE,), jnp.int32),
        pltpu.VMEM((PIECE,), jnp.int32),
        pltpu.SemaphoreType.DMA,
        pltpu.VMEM_SHARED((LISTS_LEN,), jnp.int32),
        pltpu.VMEM_SHARED((LISTS_LEN,), jnp.int32),
        pltpu.VMEM_SHARED((144,), jnp.int32),
    ],
)
def _sc_update(mem_hbm, ids_hbm, upd_hbm, out_hbm,
               win_v, idsb_v, cids_v, cqs_v, cnt1_v, counts_v, chunk_v,
               upd0_v, upd1_v, pids_v, pqs_v, presem,
               ids_sp, qs_sp, counts_sp):
  c = lax.axis_index("c")
  s = lax.axis_index("s")
  wid = c * NS + s
  f0 = wid * ROWS_PER_W
  iota16 = lax.iota(jnp.int32, 16)
  lo = s * IDS_PER_TILE

  up0 = pltpu.async_copy(upd_hbm.at[f0, :], upd0_v, presem)
  up1 = pltpu.async_copy(upd_hbm.at[f0 + 1, :], upd1_v, presem)

  neg1 = jnp.full((16,), -1, jnp.int32)

  def init_body(v, _):
    win_v[pl.ds(v * 16, 16)] = neg1
    return 0

  lax.fori_loop(0, IDS_PER_TILE // 16 + 1, init_body, 0)

  for r in range(BATCH // 4096):
    pltpu.sync_copy(ids_hbm.at[pl.ds(r * 4096, 4096)], idsb_v)

    def scan_body(v, _):
      idv = idsb_v[pl.ds(v * 16, 16)]
      kkey = (idv << P_BITS) | (iota16 + (r * 4096 + v * 16))
      idl = jnp.clip(idv - lo, 0, IDS_PER_TILE - 1)
      m_in = (idv >= lo) & (idv < lo + IDS_PER_TILE)
      plsc.store_scatter(win_v, [idl], kkey, mask=m_in)
      r0 = plsc.load_gather(win_v, [idl], mask=m_in)
      m = m_in & (kkey > r0)

      def cond(carry):
        return jnp.any(carry[0])

      def body(carry):
        (mm,) = carry
        plsc.store_scatter(win_v, [idl], kkey, mask=mm)
        r1 = plsc.load_gather(win_v, [idl], mask=m_in)
        return (m_in & (kkey > r1),)

      lax.while_loop(cond, body, (m,))
      return 0

    lax.fori_loop(0, 4096 // 16, scan_body, 0)

  def compress_body(v, cnt):
    wv = win_v[pl.ds(v * 16, 16)]
    ids16 = (lo + v * 16) + iota16
    mask = wv >= 0
    plsc.store_compressed(cids_v.at[pl.ds(cnt, 16)], ids16, mask=mask)
    plsc.store_compressed(cqs_v.at[pl.ds(cnt, 16)], wv & P_MASK, mask=mask)
    return cnt + plsc.all_reduce_population_count(mask)[0]

  def compress_tail(v, cnt):
    wv = win_v[pl.ds(v * 16, 16)]
    ids16 = (lo + v * 16) + iota16
    mask = (wv >= 0) & (iota16 < IDS_PER_TILE - 390 * 16)
    plsc.store_compressed(cids_v.at[pl.ds(cnt, 16)], ids16, mask=mask)
    plsc.store_compressed(cqs_v.at[pl.ds(cnt, 16)], wv & P_MASK, mask=mask)
    return cnt + plsc.all_reduce_population_count(mask)[0]

  cnt = lax.fori_loop(0, 390, compress_body, 0)
  cnt = compress_tail(390, cnt)

  pltpu.sync_copy(cids_v, ids_sp.at[pl.ds(s * LIST_STRIDE, LIST_STRIDE)])
  pltpu.sync_copy(cqs_v, qs_sp.at[pl.ds(s * LIST_STRIDE, LIST_STRIDE)])
  cnt1_v[pl.ds(0, 16)] = jnp.where(iota16 == 0, cnt, 0)
  pltpu.sync_copy(cnt1_v.at[pl.ds(0, 8)], counts_sp.at[pl.ds(s * 8, 8)])
  plsc.subcore_barrier()

  pltpu.sync_copy(counts_sp, counts_v)
  up0.wait()
  up1.wait()

  for rloc in range(ROWS_PER_W):
    f = f0 + rloc
    upd_row = (upd0_v, upd1_v)[rloc]
    for k in range(NCHUNK):
      c0 = k * CHUNK
      pltpu.sync_copy(mem_hbm.at[f, pl.ds(c0, CHUNK)], chunk_v)
      for j in range(4):
        t = 4 * k + j
        cntv = counts_v[pl.ds(t * 8, 16)]
        tcnt = cntv[0]
        npieces = (tcnt + PIECE - 1) // PIECE

        def piece_body(pc, _):
          base = t * LIST_STRIDE + pc * PIECE
          pltpu.sync_copy(ids_sp.at[pl.ds(base, PIECE)], pids_v)
          pltpu.sync_copy(qs_sp.at[pl.ds(base, PIECE)], pqs_v)
          rem = tcnt - pc * PIECE
          ngr = (jnp.minimum(rem, PIECE) + 15) >> 4

          def grp_body(g, _):
            idv = pids_v[pl.ds(g * 16, 16)]
            qv = pqs_v[pl.ds(g * 16, 16)]
            mask = (g * 16 + iota16) < rem
            cl = jnp.clip(idv - c0, 0, CHUNK - 1)
            u = plsc.load_gather(upd_row, [qv], mask=mask)
            cur = plsc.load_gather(chunk_v, [cl], mask=mask)
            plsc.store_scatter(chunk_v, [cl], (cur + u) * ALPHA, mask=mask)
            return 0

          lax.fori_loop(0, ngr, grp_body, 0)
          return 0

        lax.fori_loop(0, npieces, piece_body, 0)
      pltpu.sync_copy(chunk_v, out_hbm.at[f, pl.ds(c0, CHUNK)])


def kernel(mem, items_ids, items_memory):
  mem_t = jnp.swapaxes(mem, 0, 1)
  upd_t = jnp.swapaxes(items_memory, 0, 1)
  ids = items_ids.astype(jnp.int32)
  out_t = _sc_update(mem_t, ids, upd_t)
  return jnp.swapaxes(out_t, 0, 1)


# speedup vs baseline: 1.2035x; 1.0493x over previous
import jax
import jax.numpy as jnp
from jax import lax
from jax.experimental import pallas as pl
from jax.experimental.pallas import tpu as pltpu
from jax.experimental.pallas import tpu_sc as plsc

NUM_ITEMS = 100000
MEM_DIM = 64
BATCH = 16384
ALPHA = 0.5

NC = 2
NS = 16
NW = NC * NS
P_BITS = 14
P_MASK = (1 << P_BITS) - 1
IDS_PER_TILE = NUM_ITEMS // NS
WIN_PAD = 6256
LIST_STRIDE = 7168
LISTS_LEN = NS * LIST_STRIDE
CHUNK = 4 * IDS_PER_TILE
NCHUNK = NUM_ITEMS // CHUNK
ROWS_PER_W = MEM_DIM // NW
PIECE = 1024

_MESH = plsc.VectorSubcoreMesh(
    core_axis_name="c", subcore_axis_name="s", num_cores=NC, num_subcores=NS
)
_PARAMS = pltpu.CompilerParams(
    needs_layout_passes=False, use_tc_tiling_on_sc=False
)


@pl.kernel(
    out_type=jax.ShapeDtypeStruct((MEM_DIM, NUM_ITEMS), jnp.float32),
    mesh=_MESH,
    compiler_params=_PARAMS,
    scratch_types=[
        pltpu.VMEM((WIN_PAD,), jnp.int32),
        pltpu.VMEM((BATCH,), jnp.int32),
        pltpu.VMEM((LIST_STRIDE,), jnp.int32),
        pltpu.VMEM((LIST_STRIDE,), jnp.int32),
        pltpu.VMEM((16,), jnp.int32),
        pltpu.VMEM((144,), jnp.int32),
        pltpu.VMEM((CHUNK,), jnp.float32),
        pltpu.VMEM((CHUNK,), jnp.float32),
        pltpu.VMEM((BATCH,), jnp.float32),
        [pltpu.VMEM((PIECE,), jnp.int32)] * 4,
        [pltpu.VMEM((PIECE,), jnp.int32)] * 4,
        pltpu.SemaphoreType.DMA,
        pltpu.SemaphoreType.DMA,
        pltpu.SemaphoreType.DMA,
        pltpu.SemaphoreType.DMA,
        pltpu.SemaphoreType.DMA,
        pltpu.SemaphoreType.DMA,
        pltpu.VMEM_SHARED((LISTS_LEN,), jnp.int32),
        pltpu.VMEM_SHARED((LISTS_LEN,), jnp.int32),
        pltpu.VMEM_SHARED((144,), jnp.int32),
    ],
)
def _sc_update(mem_hbm, ids_hbm, upd_hbm, out_hbm,
               win_v, idsb_v, cids_v, cqs_v, cnt1_v, counts_v,
               ch0_v, ch1_v, updrow_v, pids_v, pqs_v,
               miscsem, ldsem0, ldsem1, stsem0, stsem1, piecesem,
               ids_sp, qs_sp, counts_sp):
  c = lax.axis_index("c")
  s = lax.axis_index("s")
  wid = c * NS + s
  f0 = wid * ROWS_PER_W
  iota16 = lax.iota(jnp.int32, 16)
  lo = s * IDS_PER_TILE

  chbuf = (ch0_v, ch1_v)
  ldsem = (ldsem0, ldsem1)
  stsem = (stsem0, stsem1)

  d_upd0 = pltpu.async_copy(upd_hbm.at[f0, :], updrow_v, miscsem)
  d_ch0 = pltpu.async_copy(mem_hbm.at[f0, pl.ds(0, CHUNK)], ch0_v, ldsem0)

  pltpu.sync_copy(ids_hbm, idsb_v)
  neg1 = jnp.full((16,), -1, jnp.int32)

  def init_body(v, _):
    win_v[pl.ds(v * 16, 16)] = neg1
    return 0

  lax.fori_loop(0, WIN_PAD // 16, init_body, 0)

  def plain_pass(v, _):
    idv = idsb_v[pl.ds(v * 16, 16)]
    kkey = (idv << P_BITS) | (iota16 + v * 16)
    idl = jnp.clip(idv - lo, 0, IDS_PER_TILE - 1)
    m_in = (idv >= lo) & (idv < lo + IDS_PER_TILE)
    plsc.store_scatter(win_v, [idl], kkey, mask=m_in)
    return 0

  lax.fori_loop(0, BATCH // 16, plain_pass, 0)

  def fix_pass(_):
    def body(v, acc):
      idv = idsb_v[pl.ds(v * 16, 16)]
      kkey = (idv << P_BITS) | (iota16 + v * 16)
      idl = jnp.clip(idv - lo, 0, IDS_PER_TILE - 1)
      m_in = (idv >= lo) & (idv < lo + IDS_PER_TILE)
      r0 = plsc.load_gather(win_v, [idl], mask=m_in)
      m = m_in & (kkey > r0)
      plsc.store_scatter(win_v, [idl], kkey, mask=m)
      return acc | jnp.where(m, 1, 0)

    acc = lax.fori_loop(0, BATCH // 16, body, jnp.zeros((16,), jnp.int32))
    return (jnp.any(acc > 0),)

  lax.while_loop(lambda st: st[0], lambda st: fix_pass(st), (jnp.bool_(True),))

  def compress_body(v, cnt):
    wv = win_v[pl.ds(v * 16, 16)]
    ids16 = (lo + v * 16) + iota16
    mask = (wv >= 0) & (v * 16 + iota16 < IDS_PER_TILE)
    plsc.store_compressed(cids_v.at[pl.ds(cnt, 16)], ids16, mask=mask)
    plsc.store_compressed(cqs_v.at[pl.ds(cnt, 16)], wv & P_MASK, mask=mask)
    return cnt + plsc.all_reduce_population_count(mask)[0]

  cnt = lax.fori_loop(0, WIN_PAD // 16, compress_body, 0)

  pltpu.sync_copy(cids_v, ids_sp.at[pl.ds(s * LIST_STRIDE, LIST_STRIDE)])
  pltpu.sync_copy(cqs_v, qs_sp.at[pl.ds(s * LIST_STRIDE, LIST_STRIDE)])
  cnt1_v[pl.ds(0, 16)] = jnp.where(iota16 == 0, cnt, 0)
  pltpu.sync_copy(cnt1_v.at[pl.ds(0, 8)], counts_sp.at[pl.ds(s * 8, 8)])
  plsc.subcore_barrier()

  pltpu.sync_copy(counts_sp, counts_v)

  ld = [d_ch0, None]
  st = [None, None]

  def process_shard(j, t, tcnt, chunk, c0):
    n1 = jnp.minimum(tcnt, PIECE)
    ngr = (n1 + 15) >> 4

    def grp(g, _):
      idv = pids_v[j][pl.ds(g * 16, 16)]
      qv = pqs_v[j][pl.ds(g * 16, 16)]
      mask = (g * 16 + iota16) < n1
      cl = jnp.clip(idv - c0, 0, CHUNK - 1)
      u = plsc.load_gather(updrow_v, [qv], mask=mask)
      cur = plsc.load_gather(chunk, [cl], mask=mask)
      plsc.store_scatter(chunk, [cl], (cur + u) * ALPHA, mask=mask)
      return 0

    lax.fori_loop(0, ngr, grp, 0)

    npieces = (tcnt + PIECE - 1) // PIECE

    def spill(pc, _):
      base = t * LIST_STRIDE + pc * PIECE
      pltpu.sync_copy(ids_sp.at[pl.ds(base, PIECE)], pids_v[j])
      pltpu.sync_copy(qs_sp.at[pl.ds(base, PIECE)], pqs_v[j])
      rem = tcnt - pc * PIECE
      ngr2 = (jnp.minimum(rem, PIECE) + 15) >> 4

      def grp2(g, _):
        idv = pids_v[j][pl.ds(g * 16, 16)]
        qv = pqs_v[j][pl.ds(g * 16, 16)]
        mask = (g * 16 + iota16) < rem
        cl = jnp.clip(idv - c0, 0, CHUNK - 1)
        u = plsc.load_gather(updrow_v, [qv], mask=mask)
        cur = plsc.load_gather(chunk, [cl], mask=mask)
        plsc.store_scatter(chunk, [cl], (cur + u) * ALPHA, mask=mask)
        return 0

      lax.fori_loop(0, ngr2, grp2, 0)
      return 0

    lax.fori_loop(1, npieces, spill, 0)

  for rloc in range(ROWS_PER_W):
    f = f0 + rloc
    if rloc > 0:
      pltpu.sync_copy(upd_hbm.at[f, :], updrow_v)
      if st[0] is not None:
        st[0].wait()
        st[0] = None
      ld[0] = pltpu.async_copy(mem_hbm.at[f, pl.ds(0, CHUNK)], ch0_v, ldsem0)
    else:
      d_upd0.wait()

    for k in range(NCHUNK):
      buf = k % 2
      c0 = k * CHUNK
      pdesc = []
      for j in range(4):
        t = 4 * k + j
        base = t * LIST_STRIDE
        pdesc.append(pltpu.async_copy(
            ids_sp.at[pl.ds(base, PIECE)], pids_v[j], piecesem))
        pdesc.append(pltpu.async_copy(
            qs_sp.at[pl.ds(base, PIECE)], pqs_v[j], piecesem))
      if k + 1 < NCHUNK:
        nbuf = (k + 1) % 2
        if st[nbuf] is not None:
          st[nbuf].wait()
          st[nbuf] = None
        ld[nbuf] = pltpu.async_copy(
            mem_hbm.at[f, pl.ds((k + 1) * CHUNK, CHUNK)], chbuf[nbuf],
            ldsem[nbuf])
      ld[buf].wait()
      for d in pdesc:
        d.wait()
      for j in range(4):
        t = 4 * k + j
        cntv = counts_v[pl.ds(t * 8, 16)]
        process_shard(j, t, cntv[0], chbuf[buf], c0)
      st[buf] = pltpu.async_copy(
          chbuf[buf], out_hbm.at[f, pl.ds(c0, CHUNK)], stsem[buf])

  st[0].wait()
  st[1].wait()


def kernel(mem, items_ids, items_memory):
  mem_t = jnp.swapaxes(mem, 0, 1)
  upd_t = jnp.swapaxes(items_memory, 0, 1)
  ids = items_ids.astype(jnp.int32)
  out_t = _sc_update(mem_t, ids, upd_t)
  return jnp.swapaxes(out_t, 0, 1)


# speedup vs baseline: 1.3132x; 1.0912x over previous
import jax
import jax.numpy as jnp
from jax import lax
from jax.experimental import pallas as pl
from jax.experimental.pallas import tpu as pltpu
from jax.experimental.pallas import tpu_sc as plsc

NUM_ITEMS = 100000
MEM_DIM = 64
BATCH = 16384
ALPHA = 0.5

NC = 2
NS = 16
NW = NC * NS
P_BITS = 14
P_MASK = (1 << P_BITS) - 1
IDS_PER_TILE = NUM_ITEMS // NS
WIN_PAD = 6256
LIST_STRIDE = 7168
LISTS_LEN = NS * LIST_STRIDE
CHUNK = 4 * IDS_PER_TILE
NCHUNK = NUM_ITEMS // CHUNK
ROWS_PER_W = MEM_DIM // NW
PIECE = 1024

_MESH = plsc.VectorSubcoreMesh(
    core_axis_name="c", subcore_axis_name="s", num_cores=NC, num_subcores=NS
)
_PARAMS = pltpu.CompilerParams(
    needs_layout_passes=False, use_tc_tiling_on_sc=False
)


@pl.kernel(
    out_type=jax.ShapeDtypeStruct((MEM_DIM, NUM_ITEMS), jnp.float32),
    mesh=_MESH,
    compiler_params=_PARAMS,
    scratch_types=[
        pltpu.VMEM((WIN_PAD,), jnp.int32),
        pltpu.VMEM((BATCH,), jnp.int32),
        pltpu.VMEM((LIST_STRIDE,), jnp.int32),
        pltpu.VMEM((LIST_STRIDE,), jnp.int32),
        pltpu.VMEM((16,), jnp.int32),
        pltpu.VMEM((144,), jnp.int32),
        pltpu.VMEM((CHUNK,), jnp.float32),
        pltpu.VMEM((CHUNK,), jnp.float32),
        pltpu.VMEM((BATCH,), jnp.float32),
        [pltpu.VMEM((PIECE,), jnp.int32)] * 4,
        [pltpu.VMEM((PIECE,), jnp.int32)] * 4,
        pltpu.SemaphoreType.DMA,
        pltpu.SemaphoreType.DMA,
        pltpu.SemaphoreType.DMA,
        pltpu.SemaphoreType.DMA,
        pltpu.SemaphoreType.DMA,
        pltpu.SemaphoreType.DMA,
        pltpu.VMEM_SHARED((LISTS_LEN,), jnp.int32),
        pltpu.VMEM_SHARED((LISTS_LEN,), jnp.int32),
        pltpu.VMEM_SHARED((144,), jnp.int32),
    ],
)
def _sc_update(mem_hbm, ids_hbm, upd_hbm, out_hbm,
               win_v, idsb_v, cids_v, cqs_v, cnt1_v, counts_v,
               ch0_v, ch1_v, updrow_v, pids_v, pqs_v,
               miscsem, ldsem0, ldsem1, stsem0, stsem1, piecesem,
               ids_sp, qs_sp, counts_sp):
  c = lax.axis_index("c")
  s = lax.axis_index("s")
  wid = c * NS + s
  f0 = wid * ROWS_PER_W
  iota16 = lax.iota(jnp.int32, 16)
  lo = s * IDS_PER_TILE

  chbuf = (ch0_v, ch1_v)
  ldsem = (ldsem0, ldsem1)
  stsem = (stsem0, stsem1)

  d_upd0 = pltpu.async_copy(upd_hbm.at[f0, :], updrow_v, miscsem)
  d_ch0 = pltpu.async_copy(mem_hbm.at[f0, pl.ds(0, CHUNK)], ch0_v, ldsem0)

  pltpu.sync_copy(ids_hbm, idsb_v)
  neg1 = jnp.full((16,), -1, jnp.int32)

  def init_body(v, _):
    win_v[pl.ds(v * 16, 16)] = neg1
    return 0

  lax.fori_loop(0, WIN_PAD // 16, init_body, 0)

  UNROLL = 8

  def plain_pass(v, _):
    for uu in range(UNROLL):
      p0 = v * 16 * UNROLL + uu * 16
      idv = idsb_v[pl.ds(p0, 16)]
      kkey = (idv << P_BITS) | (iota16 + p0)
      idl = jnp.clip(idv - lo, 0, IDS_PER_TILE - 1)
      m_in = (idv >= lo) & (idv < lo + IDS_PER_TILE)
      plsc.store_scatter(win_v, [idl], kkey, mask=m_in)
    return 0

  lax.fori_loop(0, BATCH // 16 // UNROLL, plain_pass, 0)

  def fix_pass(_):
    def body(v, acc):
      for uu in range(UNROLL):
        p0 = v * 16 * UNROLL + uu * 16
        idv = idsb_v[pl.ds(p0, 16)]
        kkey = (idv << P_BITS) | (iota16 + p0)
        idl = jnp.clip(idv - lo, 0, IDS_PER_TILE - 1)
        m_in = (idv >= lo) & (idv < lo + IDS_PER_TILE)
        r0 = plsc.load_gather(win_v, [idl], mask=m_in)
        m = m_in & (kkey > r0)
        plsc.store_scatter(win_v, [idl], kkey, mask=m)
        acc = acc | jnp.where(m, 1, 0)
      return acc

    acc = lax.fori_loop(0, BATCH // 16 // UNROLL, body,
                        jnp.zeros((16,), jnp.int32))
    return (jnp.any(acc > 0),)

  lax.while_loop(lambda st: st[0], lambda st: fix_pass(st), (jnp.bool_(True),))

  def compress_body(v, cnt):
    wv = win_v[pl.ds(v * 16, 16)]
    ids16 = (lo + v * 16) + iota16
    mask = (wv >= 0) & (v * 16 + iota16 < IDS_PER_TILE)
    plsc.store_compressed(cids_v.at[pl.ds(cnt, 16)], ids16, mask=mask)
    plsc.store_compressed(cqs_v.at[pl.ds(cnt, 16)], wv & P_MASK, mask=mask)
    return cnt + plsc.all_reduce_population_count(mask)[0]

  cnt = lax.fori_loop(0, WIN_PAD // 16, compress_body, 0)

  pltpu.sync_copy(cids_v, ids_sp.at[pl.ds(s * LIST_STRIDE, LIST_STRIDE)])
  pltpu.sync_copy(cqs_v, qs_sp.at[pl.ds(s * LIST_STRIDE, LIST_STRIDE)])
  cnt1_v[pl.ds(0, 16)] = jnp.where(iota16 == 0, cnt, 0)
  pltpu.sync_copy(cnt1_v.at[pl.ds(0, 8)], counts_sp.at[pl.ds(s * 8, 8)])
  plsc.subcore_barrier()

  pltpu.sync_copy(counts_sp, counts_v)

  ld = [d_ch0, None]
  st = [None, None]

  def process_shard(j, t, tcnt, chunk, c0):
    n1 = jnp.minimum(tcnt, PIECE)
    ngr = (n1 + 15) >> 4

    def grp(g, _):
      idv = pids_v[j][pl.ds(g * 16, 16)]
      qv = pqs_v[j][pl.ds(g * 16, 16)]
      mask = (g * 16 + iota16) < n1
      cl = jnp.clip(idv - c0, 0, CHUNK - 1)
      u = plsc.load_gather(updrow_v, [qv], mask=mask)
      cur = plsc.load_gather(chunk, [cl], mask=mask)
      plsc.store_scatter(chunk, [cl], (cur + u) * ALPHA, mask=mask)
      return 0

    lax.fori_loop(0, ngr, grp, 0)

    npieces = (tcnt + PIECE - 1) // PIECE

    def spill(pc, _):
      base = t * LIST_STRIDE + pc * PIECE
      pltpu.sync_copy(ids_sp.at[pl.ds(base, PIECE)], pids_v[j])
      pltpu.sync_copy(qs_sp.at[pl.ds(base, PIECE)], pqs_v[j])
      rem = tcnt - pc * PIECE
      ngr2 = (jnp.minimum(rem, PIECE) + 15) >> 4

      def grp2(g, _):
        idv = pids_v[j][pl.ds(g * 16, 16)]
        qv = pqs_v[j][pl.ds(g * 16, 16)]
        mask = (g * 16 + iota16) < rem
        cl = jnp.clip(idv - c0, 0, CHUNK - 1)
        u = plsc.load_gather(updrow_v, [qv], mask=mask)
        cur = plsc.load_gather(chunk, [cl], mask=mask)
        plsc.store_scatter(chunk, [cl], (cur + u) * ALPHA, mask=mask)
        return 0

      lax.fori_loop(0, ngr2, grp2, 0)
      return 0

    lax.fori_loop(1, npieces, spill, 0)

  for rloc in range(ROWS_PER_W):
    f = f0 + rloc
    if rloc > 0:
      pltpu.sync_copy(upd_hbm.at[f, :], updrow_v)
      if st[0] is not None:
        st[0].wait()
        st[0] = None
      ld[0] = pltpu.async_copy(mem_hbm.at[f, pl.ds(0, CHUNK)], ch0_v, ldsem0)
    else:
      d_upd0.wait()

    for k in range(NCHUNK):
      buf = k % 2
      c0 = k * CHUNK
      pdesc = []
      for j in range(4):
        t = 4 * k + j
        base = t * LIST_STRIDE
        pdesc.append(pltpu.async_copy(
            ids_sp.at[pl.ds(base, PIECE)], pids_v[j], piecesem))
        pdesc.append(pltpu.async_copy(
            qs_sp.at[pl.ds(base, PIECE)], pqs_v[j], piecesem))
      if k + 1 < NCHUNK:
        nbuf = (k + 1) % 2
        if st[nbuf] is not None:
          st[nbuf].wait()
          st[nbuf] = None
        ld[nbuf] = pltpu.async_copy(
            mem_hbm.at[f, pl.ds((k + 1) * CHUNK, CHUNK)], chbuf[nbuf],
            ldsem[nbuf])
      ld[buf].wait()
      for d in pdesc:
        d.wait()
      for j in range(4):
        t = 4 * k + j
        cntv = counts_v[pl.ds(t * 8, 16)]
        process_shard(j, t, cntv[0], chbuf[buf], c0)
      st[buf] = pltpu.async_copy(
          chbuf[buf], out_hbm.at[f, pl.ds(c0, CHUNK)], stsem[buf])

  st[0].wait()
  st[1].wait()


def kernel(mem, items_ids, items_memory):
  mem_t = jnp.swapaxes(mem, 0, 1)
  upd_t = jnp.swapaxes(items_memory, 0, 1)
  ids = items_ids.astype(jnp.int32)
  out_t = _sc_update(mem_t, ids, upd_t)
  return jnp.swapaxes(out_t, 0, 1)


# speedup vs baseline: 1.4553x; 1.1082x over previous
import jax
import jax.numpy as jnp
from jax import lax
from jax.experimental import pallas as pl
from jax.experimental.pallas import tpu as pltpu
from jax.experimental.pallas import tpu_sc as plsc

NUM_ITEMS = 100000
MEM_DIM = 64
BATCH = 16384
ALPHA = 0.5

NC = 2
NS = 16
NW = NC * NS
P_BITS = 14
P_MASK = (1 << P_BITS) - 1
IDS_PER_SHARD = NUM_ITEMS // NW
WIN_PAD = 3136
LIST_STRIDE = 4096
LISTS_LEN = NW * LIST_STRIDE
CHUNK = 8 * IDS_PER_SHARD
NCHUNK = NUM_ITEMS // CHUNK
ROWS_PER_W = MEM_DIM // NW
PIECE = 1024
SH_PER_CHUNK = CHUNK // IDS_PER_SHARD

_MESH = plsc.VectorSubcoreMesh(
    core_axis_name="c", subcore_axis_name="s", num_cores=NC, num_subcores=NS
)
_PARAMS = pltpu.CompilerParams(
    needs_layout_passes=False, use_tc_tiling_on_sc=False
)
UNROLL = 8


@pl.kernel(
    out_type=(
        jax.ShapeDtypeStruct((LISTS_LEN,), jnp.int32),
        jax.ShapeDtypeStruct((LISTS_LEN,), jnp.int32),
        jax.ShapeDtypeStruct((264,), jnp.int32),
    ),
    mesh=_MESH,
    compiler_params=_PARAMS,
    scratch_types=[
        pltpu.VMEM((WIN_PAD,), jnp.int32),
        pltpu.VMEM((BATCH,), jnp.int32),
        pltpu.VMEM((LIST_STRIDE,), jnp.int32),
        pltpu.VMEM((LIST_STRIDE,), jnp.int32),
        pltpu.VMEM((16,), jnp.int32),
    ],
)
def _sc_build(ids_hbm, idsl_hbm, qsl_hbm, counts_hbm,
              win_v, idsb_v, cids_v, cqs_v, cnt1_v):
  c = lax.axis_index("c")
  s = lax.axis_index("s")
  wid = c * NS + s
  iota16 = lax.iota(jnp.int32, 16)
  lo = wid * IDS_PER_SHARD

  pltpu.sync_copy(ids_hbm, idsb_v)
  neg1 = jnp.full((16,), -1, jnp.int32)

  def init_body(v, _):
    win_v[pl.ds(v * 16, 16)] = neg1
    return 0

  lax.fori_loop(0, WIN_PAD // 16, init_body, 0)

  def plain_pass(v, _):
    for uu in range(UNROLL):
      p0 = v * 16 * UNROLL + uu * 16
      idv = idsb_v[pl.ds(p0, 16)]
      kkey = (idv << P_BITS) | (iota16 + p0)
      idl = jnp.clip(idv - lo, 0, IDS_PER_SHARD - 1)
      m_in = (idv >= lo) & (idv < lo + IDS_PER_SHARD)
      plsc.store_scatter(win_v, [idl], kkey, mask=m_in)
    return 0

  lax.fori_loop(0, BATCH // 16 // UNROLL, plain_pass, 0)

  def fix_pass(_):
    def body(v, acc):
      for uu in range(UNROLL):
        p0 = v * 16 * UNROLL + uu * 16
        idv = idsb_v[pl.ds(p0, 16)]
        kkey = (idv << P_BITS) | (iota16 + p0)
        idl = jnp.clip(idv - lo, 0, IDS_PER_SHARD - 1)
        m_in = (idv >= lo) & (idv < lo + IDS_PER_SHARD)
        r0 = plsc.load_gather(win_v, [idl], mask=m_in)
        m = m_in & (kkey > r0)
        plsc.store_scatter(win_v, [idl], kkey, mask=m)
        acc = acc | jnp.where(m, 1, 0)
      return acc

    acc = lax.fori_loop(0, BATCH // 16 // UNROLL, body,
                        jnp.zeros((16,), jnp.int32))
    return (jnp.any(acc > 0),)

  lax.while_loop(lambda st: st[0], lambda st: fix_pass(st), (jnp.bool_(True),))

  def compress_body(v, cnt):
    wv = win_v[pl.ds(v * 16, 16)]
    ids16 = (lo + v * 16) + iota16
    mask = (wv >= 0) & (v * 16 + iota16 < IDS_PER_SHARD)
    plsc.store_compressed(cids_v.at[pl.ds(cnt, 16)], ids16, mask=mask)
    plsc.store_compressed(cqs_v.at[pl.ds(cnt, 16)], wv & P_MASK, mask=mask)
    return cnt + plsc.all_reduce_population_count(mask)[0]

  cnt = lax.fori_loop(0, WIN_PAD // 16, compress_body, 0)

  pltpu.sync_copy(cids_v, idsl_hbm.at[pl.ds(wid * LIST_STRIDE, LIST_STRIDE)])
  pltpu.sync_copy(cqs_v, qsl_hbm.at[pl.ds(wid * LIST_STRIDE, LIST_STRIDE)])
  cnt1_v[pl.ds(0, 16)] = jnp.where(iota16 == 0, cnt, 0)
  pltpu.sync_copy(cnt1_v.at[pl.ds(0, 8)], counts_hbm.at[pl.ds(wid * 8, 8)])


@pl.kernel(
    out_type=jax.ShapeDtypeStruct((MEM_DIM, NUM_ITEMS), jnp.float32),
    mesh=_MESH,
    compiler_params=_PARAMS,
    scratch_types=[
        pltpu.VMEM((264,), jnp.int32),
        pltpu.VMEM((CHUNK,), jnp.float32),
        pltpu.VMEM((CHUNK,), jnp.float32),
        pltpu.VMEM((BATCH,), jnp.float32),
        pltpu.VMEM((SH_PER_CHUNK, PIECE), jnp.int32),
        pltpu.VMEM((SH_PER_CHUNK, PIECE), jnp.int32),
        pltpu.SemaphoreType.DMA,
        pltpu.SemaphoreType.DMA,
        pltpu.SemaphoreType.DMA,
        pltpu.SemaphoreType.DMA,
        pltpu.SemaphoreType.DMA,
        pltpu.SemaphoreType.DMA,
    ],
)
def _sc_stream(mem_hbm, upd_hbm, idsl_hbm, qsl_hbm, counts_hbm, out_hbm,
               counts_v, ch0_v, ch1_v, updrow_v, pids_v, pqs_v,
               miscsem, ldsem0, ldsem1, stsem0, stsem1, piecesem):
  c = lax.axis_index("c")
  s = lax.axis_index("s")
  wid = c * NS + s
  f0 = wid * ROWS_PER_W
  iota16 = lax.iota(jnp.int32, 16)

  chbuf = (ch0_v, ch1_v)
  ldsem = (ldsem0, ldsem1)
  stsem = (stsem0, stsem1)

  d_upd0 = pltpu.async_copy(upd_hbm.at[f0, :], updrow_v, miscsem)
  pltpu.sync_copy(counts_hbm, counts_v)

  def process_chunk(k, chunk, c0):
    def shard_body(j, _):
      t = SH_PER_CHUNK * k + j
      cntv = counts_v[pl.ds(t * 8, 16)]
      tcnt = cntv[0]
      n1 = jnp.minimum(tcnt, PIECE)
      ngr = (n1 + 15) >> 4

      def grp(g, _):
        idv = pids_v[j, pl.ds(g * 16, 16)]
        qv = pqs_v[j, pl.ds(g * 16, 16)]
        mask = (g * 16 + iota16) < n1
        cl = jnp.clip(idv - c0, 0, CHUNK - 1)
        u = plsc.load_gather(updrow_v, [qv], mask=mask)
        cur = plsc.load_gather(chunk, [cl], mask=mask)
        plsc.store_scatter(chunk, [cl], (cur + u) * ALPHA, mask=mask)
        return 0

      lax.fori_loop(0, ngr, grp, 0)

      npieces = (tcnt + PIECE - 1) // PIECE

      def spill(pc, _):
        base = t * LIST_STRIDE + pc * PIECE
        pltpu.sync_copy(idsl_hbm.at[pl.ds(base, PIECE)], pids_v.at[j])
        pltpu.sync_copy(qsl_hbm.at[pl.ds(base, PIECE)], pqs_v.at[j])
        rem = tcnt - pc * PIECE
        ngr2 = (jnp.minimum(rem, PIECE) + 15) >> 4

        def grp2(g, _):
          idv = pids_v[j, pl.ds(g * 16, 16)]
          qv = pqs_v[j, pl.ds(g * 16, 16)]
          mask = (g * 16 + iota16) < rem
          cl = jnp.clip(idv - c0, 0, CHUNK - 1)
          u = plsc.load_gather(updrow_v, [qv], mask=mask)
          cur = plsc.load_gather(chunk, [cl], mask=mask)
          plsc.store_scatter(chunk, [cl], (cur + u) * ALPHA, mask=mask)
          return 0

        lax.fori_loop(0, ngr2, grp2, 0)
        return 0

      lax.fori_loop(1, npieces, spill, 0)
      return 0

    lax.fori_loop(0, SH_PER_CHUNK, shard_body, 0)

  ld = [None, None]
  st = [None, None]

  for rloc in range(ROWS_PER_W):
    f = f0 + rloc
    if rloc > 0:
      pltpu.sync_copy(upd_hbm.at[f, :], updrow_v)
      if st[0] is not None:
        st[0].wait()
        st[0] = None
    else:
      d_upd0.wait()
    ld[0] = pltpu.async_copy(mem_hbm.at[f, pl.ds(0, CHUNK)], ch0_v, ldsem0)

    for k in range(NCHUNK):
      buf = k % 2
      c0 = k * CHUNK
      pdesc = []
      for j in range(SH_PER_CHUNK):
        t = SH_PER_CHUNK * k + j
        base = t * LIST_STRIDE
        pdesc.append(pltpu.async_copy(
            idsl_hbm.at[pl.ds(base, PIECE)], pids_v.at[j], piecesem))
        pdesc.append(pltpu.async_copy(
            qsl_hbm.at[pl.ds(base, PIECE)], pqs_v.at[j], piecesem))
      if k + 1 < NCHUNK:
        nbuf = (k + 1) % 2
        if st[nbuf] is not None:
          st[nbuf].wait()
          st[nbuf] = None
        ld[nbuf] = pltpu.async_copy(
            mem_hbm.at[f, pl.ds((k + 1) * CHUNK, CHUNK)], chbuf[nbuf],
            ldsem[nbuf])
      ld[buf].wait()
      for d in pdesc:
        d.wait()
      process_chunk(k, chbuf[buf], c0)
      st[buf] = pltpu.async_copy(
          chbuf[buf], out_hbm.at[f, pl.ds(c0, CHUNK)], stsem[buf])

  st[0].wait()
  st[1].wait()


def kernel(mem, items_ids, items_memory):
  mem_t = jnp.swapaxes(mem, 0, 1)
  upd_t = jnp.swapaxes(items_memory, 0, 1)
  ids = items_ids.astype(jnp.int32)
  idsl, qsl, counts = _sc_build(ids)
  out_t = _sc_stream(mem_t, upd_t, idsl, qsl, counts)
  return jnp.swapaxes(out_t, 0, 1)
